# baseline (speedup 1.0000x reference)
import functools

import jax
import jax.numpy as jnp
import numpy as np
from jax.experimental import pallas as pl
from jax.experimental.pallas import tpu as pltpu

_DOWN = [0, 1, 2, 4, 5, 7, 8, 10, 11]
_LH = list(range(25, 40))
_RH = list(range(40, 55))
_UP = [3, 6, 9, 12, 13, 14, 15, 16, 17, 18, 19, 20, 21, 22, 23, 24]

_JC = 3
_RC = 6
_K = 512
_D = 64

_PREC = jax.lax.Precision.DEFAULT

_PARTS = ('up', 'down', 'lhand', 'rhand')
_JOINTS = {'up': _UP, 'down': _DOWN, 'lhand': _LH, 'rhand': _RH}


def _part_layout():
    offs, cps, off = {}, {}, 0
    for name in _PARTS:
        nj = len(_JOINTS[name])
        cin = nj * (_JC + _RC) + (3 if name == 'down' else 0)
        offs[name] = off
        cps[name] = nj * _JC
        off += cin
    return offs, cps, off


def _enc_row_maps():
    offs, cps, total = _part_layout()
    j2p = {}
    for name in _PARTS:
        for k, j in enumerate(_JOINTS[name]):
            j2p[j] = (name, k)
    xrows = np.zeros(55 * _JC, np.int32)
    rrows = np.zeros(55 * _RC, np.int32)
    for j in range(55):
        name, k = j2p[j]
        for cc in range(_JC):
            xrows[j * _JC + cc] = offs[name] + k * _JC + cc
        for cc in range(_RC):
            rrows[j * _RC + cc] = offs[name] + cps[name] + k * _RC + cc
    srows = np.array([offs['down'] + cps['down'] + len(_DOWN) * _RC + cc
                      for cc in range(3)], np.int32)
    return xrows, rrows, srows


def _dec_col_map(name, widths):
    joints = _JOINTS[name]
    nj = len(joints)
    cp = nj * _JC
    pos_w, rot_w = widths
    src = np.zeros(pos_w + rot_w + (3 if name == 'down' else 0), np.int32)
    msk = np.zeros_like(src, np.float32)
    for k, j in enumerate(joints):
        for cc in range(_JC):
            src[j * _JC + cc] = k * _JC + cc
            msk[j * _JC + cc] = 1.0
        for cc in range(_RC):
            src[pos_w + j * _RC + cc] = cp + k * _RC + cc
            msk[pos_w + j * _RC + cc] = 1.0
    if name == 'down':
        for cc in range(3):
            src[pos_w + rot_w + cc] = cp + nj * _RC + cc
            msk[pos_w + rot_w + cc] = 1.0
    return src, msk


def _prep_body(up_cb, dn_cb, lh_cb, rh_cb,
               up_w, dn_w, lh_w, rh_w, up_b, dn_b, lh_b, rh_b,
               t_up, t_dn, t_lh, t_rh,
               m2_up, m2_dn, m2_lh, m2_rh, cn):
    cbs = (up_cb[...], dn_cb[...], lh_cb[...], rh_cb[...])
    for cb, w, bias, tbl in zip(
            cbs, (up_w, dn_w, lh_w, rh_w), (up_b, dn_b, lh_b, rh_b),
            (t_up, t_dn, t_lh, t_rh)):
        tbl[...] = jax.lax.dot(cb, w[...], precision=_PREC) + bias[...]
    for p, (cb, m2) in enumerate(zip(cbs, (m2_up, m2_dn, m2_lh, m2_rh))):
        m2[...] = -2.0 * cb
        cn[p, :] = jnp.sum(cb * cb, axis=1)


def _main_body(x, xr, xs, wx, wr, ws, eb,
               m2_up, m2_dn, m2_lh, m2_rh, cn,
               t_up, t_dn, t_lh, t_rh,
               xout, xrout, xshift, loss, *, tile, n_rows):
    xt = x[0]
    xrt = xr[0]
    xst = xs[0]
    z_all = (jax.lax.dot(xt, wx[...], precision=_PREC)
             + jax.lax.dot(xrt, wr[...], precision=_PREC)
             + jax.lax.dot(xst, ws[...], precision=_PREC)
             + eb[...])

    cn_all = cn[...]
    dmin_total = jnp.sum(z_all * z_all)
    iota = jax.lax.broadcasted_iota(jnp.int32, (tile, _K), 1).astype(
        jnp.float32)
    dec = []
    for p, (m2cb, tbl) in enumerate(
            ((m2_up, t_up), (m2_dn, t_dn), (m2_lh, t_lh), (m2_rh, t_rh))):
        z = z_all[:, p * _D:(p + 1) * _D]
        e = jax.lax.dot_general(z, m2cb[...], (((1,), (1,)), ((), ())),
                                precision=_PREC) + cn_all[p, :][None, :]
        m = jnp.min(e, axis=1)
        dmin_total += jnp.sum(m)
        cand = jnp.where(e == m[:, None], iota, jnp.float32(_K))
        idx = jnp.min(cand, axis=1)
        oh = (iota == idx[:, None]).astype(jnp.float32)
        dec.append(jax.lax.dot(oh, tbl[...], precision=_PREC))
    d_up, d_dn, d_lh, d_rh = dec

    xout[...] = jnp.concatenate(
        [d_up[:, :36] + d_dn[:, :36], d_up[:, 36:75], xt[:, 75:165]],
        axis=1)[None]
    xrout[...] = jnp.concatenate(
        [d_up[:, 75:147] + d_dn[:, 36:108], d_up[:, 147:225],
         d_lh, d_rh], axis=1)[None]
    xshift[...] = d_dn[:, 108:111][None]

    @pl.when((pl.program_id(0) == 0) & (pl.program_id(1) == 0))
    def _():
        loss[...] = jnp.zeros_like(loss)
    loss[...] += (1.25 / (n_rows * _D)) * dmin_total.reshape(1, 1)


def kernel(x, xrot, xshift, up_encW, up_encb, up_codebook, up_decW, up_decb,
           down_encW, down_encb, down_codebook, down_decW, down_decb,
           lhand_encW, lhand_encb, lhand_codebook, lhand_decW, lhand_decb,
           rhand_encW, rhand_encb, rhand_codebook, rhand_decW, rhand_decb):
    b, t, c = x.shape
    crot = xrot.shape[-1]
    n = b * t

    wfull = jax.scipy.linalg.block_diag(up_encW, down_encW, lhand_encW,
                                        rhand_encW)
    xrows, rrows, srows = _enc_row_maps()
    wx = wfull[xrows]
    wr = wfull[rrows]
    ws = wfull[srows]
    eb = jnp.concatenate([up_encb, down_encb, lhand_encb,
                          rhand_encb])[None, :]

    up_src, up_msk = _dec_col_map('up', (75, 150))
    dn_src, dn_msk = _dec_col_map('down', (36, 72))
    up_w = up_decW[:, up_src] * up_msk[None, :]
    up_b = (up_decb[up_src] * up_msk)[None, :]
    dn_w = down_decW[:, dn_src] * dn_msk[None, :]
    dn_b = (down_decb[dn_src] * dn_msk)[None, :]
    lh_cp = len(_LH) * _JC
    lh_w = lhand_decW[:, lh_cp:lh_cp + len(_LH) * _RC]
    lh_b = lhand_decb[None, lh_cp:lh_cp + len(_LH) * _RC]
    rh_cp = len(_RH) * _JC
    rh_w = rhand_decW[:, rh_cp:rh_cp + len(_RH) * _RC]
    rh_b = rhand_decb[None, rh_cp:rh_cp + len(_RH) * _RC]

    tbl_shapes = [
        jax.ShapeDtypeStruct((_K, 225), jnp.float32),
        jax.ShapeDtypeStruct((_K, 111), jnp.float32),
        jax.ShapeDtypeStruct((_K, 90), jnp.float32),
        jax.ShapeDtypeStruct((_K, 90), jnp.float32),
        jax.ShapeDtypeStruct((_K, _D), jnp.float32),
        jax.ShapeDtypeStruct((_K, _D), jnp.float32),
        jax.ShapeDtypeStruct((_K, _D), jnp.float32),
        jax.ShapeDtypeStruct((_K, _D), jnp.float32),
        jax.ShapeDtypeStruct((4, _K), jnp.float32),
    ]
    prep = pl.pallas_call(
        _prep_body,
        out_shape=tbl_shapes,
    )(up_codebook, down_codebook, lhand_codebook, rhand_codebook,
      up_w, dn_w, lh_w, rh_w, up_b, dn_b, lh_b, rh_b)
    t_up, t_dn, t_lh, t_rh, m2_up, m2_dn, m2_lh, m2_rh, cn = prep

    tile = 256
    grid = (b, t // tile)
    blk = lambda w: pl.BlockSpec((1, tile, w), lambda i, j: (i, j, 0))
    full = lambda a: pl.BlockSpec(a.shape, lambda i, j: (0,) * a.ndim)

    out_shapes = [
        jax.ShapeDtypeStruct((b, t, c), jnp.float32),
        jax.ShapeDtypeStruct((b, t, crot), jnp.float32),
        jax.ShapeDtypeStruct((b, t, 3), jnp.float32),
        jax.ShapeDtypeStruct((1, 1), jnp.float32),
    ]
    outs = pl.pallas_call(
        functools.partial(_main_body, tile=tile, n_rows=n),
        grid=grid,
        in_specs=[blk(c), blk(crot), blk(3),
                  full(wx), full(wr), full(ws), full(eb),
                  full(m2_up), full(m2_dn), full(m2_lh), full(m2_rh),
                  full(cn),
                  full(t_up), full(t_dn), full(t_lh), full(t_rh)],
        out_specs=[blk(c), blk(crot), blk(3),
                   pl.BlockSpec((1, 1), lambda i, j: (0, 0))],
        out_shape=out_shapes,
    )(x, xrot, xshift, wx, wr, ws, eb,
      m2_up, m2_dn, m2_lh, m2_rh, cn, t_up, t_dn, t_lh, t_rh)
    xout3, xrout3, xshift3, loss = outs

    return (xout3, xrout3, xshift3, loss[0, 0])


# speedup vs baseline: 1.1382x; 1.1382x over previous
import functools

import jax
import jax.numpy as jnp
import numpy as np
from jax.experimental import pallas as pl
from jax.experimental.pallas import tpu as pltpu

_DOWN = [0, 1, 2, 4, 5, 7, 8, 10, 11]
_LH = list(range(25, 40))
_RH = list(range(40, 55))
_UP = [3, 6, 9, 12, 13, 14, 15, 16, 17, 18, 19, 20, 21, 22, 23, 24]

_JC = 3
_RC = 6
_K = 512
_D = 64

_PREC = jax.lax.Precision.DEFAULT

_PARTS = ('up', 'down', 'lhand', 'rhand')
_JOINTS = {'up': _UP, 'down': _DOWN, 'lhand': _LH, 'rhand': _RH}


def _part_layout():
    offs, cps, off = {}, {}, 0
    for name in _PARTS:
        nj = len(_JOINTS[name])
        cin = nj * (_JC + _RC) + (3 if name == 'down' else 0)
        offs[name] = off
        cps[name] = nj * _JC
        off += cin
    return offs, cps, off


def _enc_row_maps():
    offs, cps, total = _part_layout()
    j2p = {}
    for name in _PARTS:
        for k, j in enumerate(_JOINTS[name]):
            j2p[j] = (name, k)
    xrows = np.zeros(55 * _JC, np.int32)
    rrows = np.zeros(55 * _RC, np.int32)
    for j in range(55):
        name, k = j2p[j]
        for cc in range(_JC):
            xrows[j * _JC + cc] = offs[name] + k * _JC + cc
        for cc in range(_RC):
            rrows[j * _RC + cc] = offs[name] + cps[name] + k * _RC + cc
    srows = np.array([offs['down'] + cps['down'] + len(_DOWN) * _RC + cc
                      for cc in range(3)], np.int32)
    return xrows, rrows, srows


def _dec_col_map(name, widths):
    joints = _JOINTS[name]
    nj = len(joints)
    cp = nj * _JC
    pos_w, rot_w = widths
    src = np.zeros(pos_w + rot_w + (3 if name == 'down' else 0), np.int32)
    msk = np.zeros_like(src, np.float32)
    for k, j in enumerate(joints):
        for cc in range(_JC):
            src[j * _JC + cc] = k * _JC + cc
            msk[j * _JC + cc] = 1.0
        for cc in range(_RC):
            src[pos_w + j * _RC + cc] = cp + k * _RC + cc
            msk[pos_w + j * _RC + cc] = 1.0
    if name == 'down':
        for cc in range(3):
            src[pos_w + rot_w + cc] = cp + nj * _RC + cc
            msk[pos_w + rot_w + cc] = 1.0
    return src, msk


def _prep_body(up_cb, dn_cb, lh_cb, rh_cb,
               up_w, dn_w, lh_w, rh_w, up_b, dn_b, lh_b, rh_b,
               t_up, t_dn, t_lh, t_rh,
               m2_up, m2_dn, m2_lh, m2_rh, cn):
    cbs = (up_cb[...], dn_cb[...], lh_cb[...], rh_cb[...])
    for cb, w, bias, tbl in zip(
            cbs, (up_w, dn_w, lh_w, rh_w), (up_b, dn_b, lh_b, rh_b),
            (t_up, t_dn, t_lh, t_rh)):
        tbl[...] = jax.lax.dot(cb, w[...], precision=_PREC) + bias[...]
    for p, (cb, m2) in enumerate(zip(cbs, (m2_up, m2_dn, m2_lh, m2_rh))):
        m2[...] = -2.0 * cb
        cn[p, :] = jnp.sum(cb * cb, axis=1)


def _main_body(x, xr, xs, wx, wr, ws, eb,
               m2_up, m2_dn, m2_lh, m2_rh, cn,
               t_up, t_dn, t_lh, t_rh,
               xout, xrout, xshift, loss, *, tile, n_rows):
    xt = x[0]
    xrt = xr[0]
    xst = xs[0]
    z_all = (jax.lax.dot(xt, wx[...], precision=_PREC)
             + jax.lax.dot(xrt, wr[...], precision=_PREC)
             + jax.lax.dot(xst, ws[...], precision=_PREC)
             + eb[...])

    cn_all = cn[...]
    dmin_total = jnp.sum(z_all * z_all)
    iota = jax.lax.broadcasted_iota(jnp.int32, (tile, _K), 1).astype(
        jnp.float32)
    dec = []
    for p, (m2cb, tbl) in enumerate(
            ((m2_up, t_up), (m2_dn, t_dn), (m2_lh, t_lh), (m2_rh, t_rh))):
        z = z_all[:, p * _D:(p + 1) * _D]
        e = jax.lax.dot_general(z, m2cb[...], (((1,), (1,)), ((), ())),
                                precision=_PREC) + cn_all[p, :][None, :]
        m = jnp.min(e, axis=1)
        dmin_total += jnp.sum(m)
        cand = jnp.where(e == m[:, None], iota, jnp.float32(_K))
        idx = jnp.min(cand, axis=1)
        oh = (iota == idx[:, None]).astype(jnp.float32)
        dec.append(jax.lax.dot(oh, tbl[...], precision=_PREC))
    d_up, d_dn, d_lh, d_rh = dec

    xout[...] = jnp.concatenate(
        [d_up[:, :36] + d_dn[:, :36], d_up[:, 36:75], xt[:, 75:165]],
        axis=1)[None]
    xrout[...] = jnp.concatenate(
        [d_up[:, 75:147] + d_dn[:, 36:108], d_up[:, 147:225],
         d_lh, d_rh], axis=1)[None]
    xshift[...] = d_dn[:, 108:111][None]

    @pl.when((pl.program_id(0) == 0) & (pl.program_id(1) == 0))
    def _():
        loss[...] = jnp.zeros_like(loss)
    loss[...] += (1.25 / (n_rows * _D)) * dmin_total.reshape(1, 1)


def kernel(x, xrot, xshift, up_encW, up_encb, up_codebook, up_decW, up_decb,
           down_encW, down_encb, down_codebook, down_decW, down_decb,
           lhand_encW, lhand_encb, lhand_codebook, lhand_decW, lhand_decb,
           rhand_encW, rhand_encb, rhand_codebook, rhand_decW, rhand_decb):
    b, t, c = x.shape
    crot = xrot.shape[-1]
    n = b * t

    wfull = jax.scipy.linalg.block_diag(up_encW, down_encW, lhand_encW,
                                        rhand_encW)
    xrows, rrows, srows = _enc_row_maps()
    wx = wfull[xrows]
    wr = wfull[rrows]
    ws = wfull[srows]
    eb = jnp.concatenate([up_encb, down_encb, lhand_encb,
                          rhand_encb])[None, :]

    up_src, up_msk = _dec_col_map('up', (75, 150))
    dn_src, dn_msk = _dec_col_map('down', (36, 72))
    up_w = up_decW[:, up_src] * up_msk[None, :]
    up_b = (up_decb[up_src] * up_msk)[None, :]
    dn_w = down_decW[:, dn_src] * dn_msk[None, :]
    dn_b = (down_decb[dn_src] * dn_msk)[None, :]
    lh_cp = len(_LH) * _JC
    lh_w = lhand_decW[:, lh_cp:lh_cp + len(_LH) * _RC]
    lh_b = lhand_decb[None, lh_cp:lh_cp + len(_LH) * _RC]
    rh_cp = len(_RH) * _JC
    rh_w = rhand_decW[:, rh_cp:rh_cp + len(_RH) * _RC]
    rh_b = rhand_decb[None, rh_cp:rh_cp + len(_RH) * _RC]

    tbl_shapes = [
        jax.ShapeDtypeStruct((_K, 225), jnp.float32),
        jax.ShapeDtypeStruct((_K, 111), jnp.float32),
        jax.ShapeDtypeStruct((_K, 90), jnp.float32),
        jax.ShapeDtypeStruct((_K, 90), jnp.float32),
        jax.ShapeDtypeStruct((_K, _D), jnp.float32),
        jax.ShapeDtypeStruct((_K, _D), jnp.float32),
        jax.ShapeDtypeStruct((_K, _D), jnp.float32),
        jax.ShapeDtypeStruct((_K, _D), jnp.float32),
        jax.ShapeDtypeStruct((4, _K), jnp.float32),
    ]
    prep = pl.pallas_call(
        _prep_body,
        out_shape=tbl_shapes,
    )(up_codebook, down_codebook, lhand_codebook, rhand_codebook,
      up_w, dn_w, lh_w, rh_w, up_b, dn_b, lh_b, rh_b)
    t_up, t_dn, t_lh, t_rh, m2_up, m2_dn, m2_lh, m2_rh, cn = prep

    tile = 1024
    grid = (b, t // tile)
    blk = lambda w: pl.BlockSpec((1, tile, w), lambda i, j: (i, j, 0))
    full = lambda a: pl.BlockSpec(a.shape, lambda i, j: (0,) * a.ndim)

    out_shapes = [
        jax.ShapeDtypeStruct((b, t, c), jnp.float32),
        jax.ShapeDtypeStruct((b, t, crot), jnp.float32),
        jax.ShapeDtypeStruct((b, t, 3), jnp.float32),
        jax.ShapeDtypeStruct((1, 1), jnp.float32),
    ]
    outs = pl.pallas_call(
        functools.partial(_main_body, tile=tile, n_rows=n),
        grid=grid,
        in_specs=[blk(c), blk(crot), blk(3),
                  full(wx), full(wr), full(ws), full(eb),
                  full(m2_up), full(m2_dn), full(m2_lh), full(m2_rh),
                  full(cn),
                  full(t_up), full(t_dn), full(t_lh), full(t_rh)],
        out_specs=[blk(c), blk(crot), blk(3),
                   pl.BlockSpec((1, 1), lambda i, j: (0, 0))],
        out_shape=out_shapes,
    )(x, xrot, xshift, wx, wr, ws, eb,
      m2_up, m2_dn, m2_lh, m2_rh, cn, t_up, t_dn, t_lh, t_rh)
    xout3, xrout3, xshift3, loss = outs

    return (xout3, xrout3, xshift3, loss[0, 0])


# speedup vs baseline: 1.1628x; 1.0216x over previous
import functools

import jax
import jax.numpy as jnp
import numpy as np
from jax.experimental import pallas as pl
from jax.experimental.pallas import tpu as pltpu

_DOWN = [0, 1, 2, 4, 5, 7, 8, 10, 11]
_LH = list(range(25, 40))
_RH = list(range(40, 55))
_UP = [3, 6, 9, 12, 13, 14, 15, 16, 17, 18, 19, 20, 21, 22, 23, 24]

_JC = 3
_RC = 6
_K = 512
_D = 64

_PREC = jax.lax.Precision.DEFAULT

_PARTS = ('up', 'down', 'lhand', 'rhand')
_JOINTS = {'up': _UP, 'down': _DOWN, 'lhand': _LH, 'rhand': _RH}


def _part_layout():
    offs, cps, off = {}, {}, 0
    for name in _PARTS:
        nj = len(_JOINTS[name])
        cin = nj * (_JC + _RC) + (3 if name == 'down' else 0)
        offs[name] = off
        cps[name] = nj * _JC
        off += cin
    return offs, cps, off


def _enc_row_maps():
    offs, cps, total = _part_layout()
    j2p = {}
    for name in _PARTS:
        for k, j in enumerate(_JOINTS[name]):
            j2p[j] = (name, k)
    xrows = np.zeros(55 * _JC, np.int32)
    rrows = np.zeros(55 * _RC, np.int32)
    for j in range(55):
        name, k = j2p[j]
        for cc in range(_JC):
            xrows[j * _JC + cc] = offs[name] + k * _JC + cc
        for cc in range(_RC):
            rrows[j * _RC + cc] = offs[name] + cps[name] + k * _RC + cc
    srows = np.array([offs['down'] + cps['down'] + len(_DOWN) * _RC + cc
                      for cc in range(3)], np.int32)
    return xrows, rrows, srows


def _dec_col_map(name, widths):
    joints = _JOINTS[name]
    nj = len(joints)
    cp = nj * _JC
    pos_w, rot_w = widths
    src = np.zeros(pos_w + rot_w + (3 if name == 'down' else 0), np.int32)
    msk = np.zeros_like(src, np.float32)
    for k, j in enumerate(joints):
        for cc in range(_JC):
            src[j * _JC + cc] = k * _JC + cc
            msk[j * _JC + cc] = 1.0
        for cc in range(_RC):
            src[pos_w + j * _RC + cc] = cp + k * _RC + cc
            msk[pos_w + j * _RC + cc] = 1.0
    if name == 'down':
        for cc in range(3):
            src[pos_w + rot_w + cc] = cp + nj * _RC + cc
            msk[pos_w + rot_w + cc] = 1.0
    return src, msk


def _prep_body(up_cb, dn_cb, lh_cb, rh_cb,
               up_w, dn_w, lh_w, rh_w, up_b, dn_b, lh_b, rh_b,
               t_up, t_dn, t_lh, t_rh,
               m2_up, m2_dn, m2_lh, m2_rh, cn):
    cbs = (up_cb[...], dn_cb[...], lh_cb[...], rh_cb[...])
    for cb, w, bias, tbl in zip(
            cbs, (up_w, dn_w, lh_w, rh_w), (up_b, dn_b, lh_b, rh_b),
            (t_up, t_dn, t_lh, t_rh)):
        tbl[...] = jax.lax.dot(cb, w[...], precision=_PREC) + bias[...]
    for p, (cb, m2) in enumerate(zip(cbs, (m2_up, m2_dn, m2_lh, m2_rh))):
        m2[...] = -2.0 * cb
        cn[p, :] = jnp.sum(cb * cb, axis=1)


def _main_body(x, xr, xs, wx, wr, ws, eb,
               m2_up, m2_dn, m2_lh, m2_rh, cn,
               t_up, t_dn, t_lh, t_rh,
               xout, xrout, xshift, loss, *, tile, n_rows):
    xt = x[0]
    xrt = xr[0]
    xst = xs[0]
    z_all = (jax.lax.dot(xt, wx[...], precision=_PREC)
             + jax.lax.dot(xrt, wr[...], precision=_PREC)
             + jax.lax.dot(xst, ws[...], precision=_PREC)
             + eb[...])

    cn_all = cn[...]
    dmin_total = jnp.sum(z_all * z_all)
    iota = jax.lax.broadcasted_iota(jnp.int32, (tile, _K), 1).astype(
        jnp.float32)
    dec = []
    for p, (m2cb, tbl) in enumerate(
            ((m2_up, t_up), (m2_dn, t_dn), (m2_lh, t_lh), (m2_rh, t_rh))):
        z = z_all[:, p * _D:(p + 1) * _D]
        e = jax.lax.dot_general(z, m2cb[...], (((1,), (1,)), ((), ())),
                                precision=_PREC) + cn_all[p, :][None, :]
        m = jnp.min(e, axis=1)
        dmin_total += jnp.sum(m)
        cand = jnp.where(e == m[:, None], iota, jnp.float32(_K))
        idx = jnp.min(cand, axis=1)
        oh = (iota == idx[:, None]).astype(jnp.float32)
        dec.append(jax.lax.dot(oh, tbl[...], precision=_PREC))
    d_up, d_dn, d_lh, d_rh = dec

    xout[...] = jnp.concatenate(
        [d_up[:, :36] + d_dn[:, :36], d_up[:, 36:75], xt[:, 75:165]],
        axis=1)[None]
    xrout[...] = jnp.concatenate(
        [d_up[:, 75:147] + d_dn[:, 36:108], d_up[:, 147:225],
         d_lh, d_rh], axis=1)[None]
    xshift[...] = d_dn[:, 108:111][None]

    @pl.when((pl.program_id(0) == 0) & (pl.program_id(1) == 0))
    def _():
        loss[...] = jnp.zeros_like(loss)
    loss[...] += (1.25 / (n_rows * _D)) * dmin_total.reshape(1, 1)


def kernel(x, xrot, xshift, up_encW, up_encb, up_codebook, up_decW, up_decb,
           down_encW, down_encb, down_codebook, down_decW, down_decb,
           lhand_encW, lhand_encb, lhand_codebook, lhand_decW, lhand_decb,
           rhand_encW, rhand_encb, rhand_codebook, rhand_decW, rhand_decb):
    b, t, c = x.shape
    crot = xrot.shape[-1]
    n = b * t

    wfull = jax.scipy.linalg.block_diag(up_encW, down_encW, lhand_encW,
                                        rhand_encW)
    xrows, rrows, srows = _enc_row_maps()
    wx = wfull[xrows]
    wr = wfull[rrows]
    ws = wfull[srows]
    eb = jnp.concatenate([up_encb, down_encb, lhand_encb,
                          rhand_encb])[None, :]

    up_src, up_msk = _dec_col_map('up', (75, 150))
    dn_src, dn_msk = _dec_col_map('down', (36, 72))
    up_w = up_decW[:, up_src] * up_msk[None, :]
    up_b = (up_decb[up_src] * up_msk)[None, :]
    dn_w = down_decW[:, dn_src] * dn_msk[None, :]
    dn_b = (down_decb[dn_src] * dn_msk)[None, :]
    lh_cp = len(_LH) * _JC
    lh_w = lhand_decW[:, lh_cp:lh_cp + len(_LH) * _RC]
    lh_b = lhand_decb[None, lh_cp:lh_cp + len(_LH) * _RC]
    rh_cp = len(_RH) * _JC
    rh_w = rhand_decW[:, rh_cp:rh_cp + len(_RH) * _RC]
    rh_b = rhand_decb[None, rh_cp:rh_cp + len(_RH) * _RC]

    tbl_shapes = [
        jax.ShapeDtypeStruct((_K, 225), jnp.float32),
        jax.ShapeDtypeStruct((_K, 111), jnp.float32),
        jax.ShapeDtypeStruct((_K, 90), jnp.float32),
        jax.ShapeDtypeStruct((_K, 90), jnp.float32),
        jax.ShapeDtypeStruct((_K, _D), jnp.float32),
        jax.ShapeDtypeStruct((_K, _D), jnp.float32),
        jax.ShapeDtypeStruct((_K, _D), jnp.float32),
        jax.ShapeDtypeStruct((_K, _D), jnp.float32),
        jax.ShapeDtypeStruct((4, _K), jnp.float32),
    ]
    prep = pl.pallas_call(
        _prep_body,
        out_shape=tbl_shapes,
    )(up_codebook, down_codebook, lhand_codebook, rhand_codebook,
      up_w, dn_w, lh_w, rh_w, up_b, dn_b, lh_b, rh_b)
    t_up, t_dn, t_lh, t_rh, m2_up, m2_dn, m2_lh, m2_rh, cn = prep

    tile = 512
    grid = (b, t // tile)
    blk = lambda w: pl.BlockSpec((1, tile, w), lambda i, j: (i, j, 0))
    full = lambda a: pl.BlockSpec(a.shape, lambda i, j: (0,) * a.ndim)

    out_shapes = [
        jax.ShapeDtypeStruct((b, t, c), jnp.float32),
        jax.ShapeDtypeStruct((b, t, crot), jnp.float32),
        jax.ShapeDtypeStruct((b, t, 3), jnp.float32),
        jax.ShapeDtypeStruct((1, 1), jnp.float32),
    ]
    outs = pl.pallas_call(
        functools.partial(_main_body, tile=tile, n_rows=n),
        grid=grid,
        in_specs=[blk(c), blk(crot), blk(3),
                  full(wx), full(wr), full(ws), full(eb),
                  full(m2_up), full(m2_dn), full(m2_lh), full(m2_rh),
                  full(cn),
                  full(t_up), full(t_dn), full(t_lh), full(t_rh)],
        out_specs=[blk(c), blk(crot), blk(3),
                   pl.BlockSpec((1, 1), lambda i, j: (0, 0))],
        out_shape=out_shapes,
    )(x, xrot, xshift, wx, wr, ws, eb,
      m2_up, m2_dn, m2_lh, m2_rh, cn, t_up, t_dn, t_lh, t_rh)
    xout3, xrout3, xshift3, loss = outs

    return (xout3, xrout3, xshift3, loss[0, 0])
